# baseline (speedup 1.0000x reference)
import functools

import jax
import jax.numpy as jnp
from jax import lax
from jax.experimental import pallas as pl
from jax.experimental.pallas import tpu as pltpu
from jax.experimental.pallas import tpu_sc as plsc

_B, _D, _N = 32, 256, 576
_K = 1024
_H = _B // 2

_info = plsc.get_sparse_core_info()
_NC, _NS = _info.num_cores, _info.num_subcores
_NW = _NC * _NS
_RPW = (_H * _N) // _NW
_TB = 4


def _argmin_body(x_ref, embed_ref, x2_ref, e2_ref, idx_ref):
    xb = x_ref[0]
    scores = lax.dot_general(
        embed_ref[...], xb,
        dimension_numbers=(((1,), (0,)), ((), ())),
        precision=lax.Precision.DEFAULT,
        preferred_element_type=jnp.float32)
    q = (x2_ref[0] - 2.0 * scores) + e2_ref[...]
    minq = jnp.min(q, axis=0, keepdims=True)
    mc = jnp.maximum(minq, 0.0)
    sm = jnp.sqrt(mc)
    mi = lax.bitcast_convert_type(mc, jnp.int32)
    h = mc
    for delta in (1, 2, 3, 4):
        c = lax.bitcast_convert_type(mi + delta, jnp.float32)
        h = jnp.where(jnp.sqrt(c) == sm, c, h)
    iota = lax.broadcasted_iota(jnp.int32, (_K, _N), 0).astype(jnp.float32)
    cand = jnp.where(q <= h, iota, float(_K))
    idx_ref[0] = jnp.min(cand, axis=0, keepdims=True).astype(jnp.int32)


def _codes_half(xh, embed, x2h, e2):
    idx3 = pl.pallas_call(
        _argmin_body,
        grid=(_H,),
        in_specs=[
            pl.BlockSpec((1, _D, _N), lambda b: (b, 0, 0)),
            pl.BlockSpec((_K, _D), lambda b: (0, 0)),
            pl.BlockSpec((1, 1, _N), lambda b: (b, 0, 0)),
            pl.BlockSpec((_K, 1), lambda b: (0, 0)),
        ],
        out_specs=pl.BlockSpec((1, 1, _N), lambda b: (b, 0, 0)),
        out_shape=jax.ShapeDtypeStruct((_H, 1, _N), jnp.int32),
    )(xh, embed, x2h, e2)
    return idx3.reshape(_H, _N)


@functools.partial(
    pl.kernel,
    mesh=plsc.VectorSubcoreMesh(core_axis_name="c", subcore_axis_name="s"),
    out_type=jax.ShapeDtypeStruct((_H * _N, _D), jnp.float32),
    scratch_types=[
        pltpu.VMEM((_RPW,), jnp.int32),
        pltpu.VMEM((_RPW, _D), jnp.float32),
        pltpu.SemaphoreType.DMA,
    ],
)
def _sc_gather(table_hbm, idx_hbm, out_hbm, idx_v, rows_v, sem):
    wid = lax.axis_index("s") * _NC + lax.axis_index("c")
    base = wid * _RPW
    pltpu.sync_copy(idx_hbm.at[pl.ds(base, _RPW)], idx_v)
    pltpu.async_copy(table_hbm.at[idx_v], rows_v, sem).wait()
    pltpu.sync_copy(rows_v, out_hbm.at[pl.ds(base, _RPW)])


def _transpose_body(rows_ref, out_ref):
    for b in range(_TB):
        out_ref[b] = rows_ref[b].T


def _transpose_half(rows):
    return pl.pallas_call(
        _transpose_body,
        grid=(_H // _TB,),
        in_specs=[pl.BlockSpec((_TB, _N, _D), lambda b: (b, 0, 0))],
        out_specs=pl.BlockSpec((_TB, _D, _N), lambda b: (b, 0, 0)),
        out_shape=jax.ShapeDtypeStruct((_H, _D, _N), jnp.float32),
    )(rows)


def kernel(x, embed):
    x2 = jnp.sum(x * x, axis=1).reshape(_B, 1, _N)
    e2 = jnp.sum(embed * embed, axis=-1).reshape(_K, 1)

    ind0 = _codes_half(x[:_H], embed, x2[:_H], e2)
    ind1 = _codes_half(x[_H:], embed, x2[_H:], e2)
    rows0 = _sc_gather(embed, ind0.reshape(_H * _N))
    rows1 = _sc_gather(embed, ind1.reshape(_H * _N))
    q0 = _transpose_half(rows0.reshape(_H, _N, _D))
    q1 = _transpose_half(rows1.reshape(_H, _N, _D))
    quantize = jnp.concatenate([q0, q1], axis=0)
    embed_ind = jnp.concatenate([ind0, ind1], axis=0)
    return (quantize, embed_ind)


# speedup vs baseline: 1.2071x; 1.2071x over previous
import functools

import jax
import jax.numpy as jnp
from jax import lax
from jax.experimental import pallas as pl
from jax.experimental.pallas import tpu as pltpu
from jax.experimental.pallas import tpu_sc as plsc

_B, _D, _N = 32, 256, 576
_K = 1024
_NP = 640

_info = plsc.get_sparse_core_info()
_NC, _NS = _info.num_cores, _info.num_subcores
_NW = _NC * _NS
_RPW = (_B * _N) // _NW
_CHUNK = 288
_TB = 4


def _argmin_body(x_ref, embed_ref, x2_ref, e2_ref, idx_ref):
    xb = x_ref[0]
    scores = lax.dot_general(
        embed_ref[...], xb,
        dimension_numbers=(((1,), (0,)), ((), ())),
        precision=lax.Precision.DEFAULT,
        preferred_element_type=jnp.float32)
    q = (x2_ref[0] - 2.0 * scores) + e2_ref[...]
    minq = jnp.min(q, axis=0, keepdims=True)
    mc = jnp.maximum(minq, 0.0)
    sm = jnp.sqrt(mc)
    mi = lax.bitcast_convert_type(mc, jnp.int32)
    h = mc
    for delta in (1, 2, 3, 4):
        c = lax.bitcast_convert_type(mi + delta, jnp.float32)
        h = jnp.where(jnp.sqrt(c) == sm, c, h)
    iota = lax.broadcasted_iota(jnp.int32, (_K, _N), 0).astype(jnp.float32)
    cand = jnp.where(q <= h, iota, float(_K))
    idx_ref[0] = jnp.min(cand, axis=0, keepdims=True).astype(jnp.int32)


def _compute_codes(x, embed, x2, e2):
    idx3 = pl.pallas_call(
        _argmin_body,
        grid=(_B,),
        in_specs=[
            pl.BlockSpec((1, _D, _N), lambda b: (b, 0, 0)),
            pl.BlockSpec((_K, _D), lambda b: (0, 0)),
            pl.BlockSpec((1, 1, _N), lambda b: (b, 0, 0)),
            pl.BlockSpec((_K, 1), lambda b: (0, 0)),
        ],
        out_specs=pl.BlockSpec((1, 1, _N), lambda b: (b, 0, 0)),
        out_shape=jax.ShapeDtypeStruct((_B, 1, _N), jnp.int32),
    )(x, embed, x2, e2)
    return idx3.reshape(_B, _N)


@functools.partial(
    pl.kernel,
    mesh=plsc.VectorSubcoreMesh(core_axis_name="c", subcore_axis_name="s"),
    out_type=jax.ShapeDtypeStruct((_B * _N, _D), jnp.float32),
    scratch_types=[
        pltpu.VMEM((_RPW,), jnp.int32),
        pltpu.VMEM((_CHUNK, _D), jnp.float32),
        pltpu.SemaphoreType.DMA,
    ],
)
def _sc_gather(table_hbm, idx_hbm, out_hbm, idx_v, rows_v, sem):
    wid = lax.axis_index("s") * _NC + lax.axis_index("c")
    base = wid * _RPW
    pltpu.sync_copy(idx_hbm.at[pl.ds(base, _RPW)], idx_v)
    for c in range(_RPW // _CHUNK):
        pltpu.async_copy(
            table_hbm.at[idx_v.at[pl.ds(c * _CHUNK, _CHUNK)]], rows_v, sem
        ).wait()
        pltpu.sync_copy(rows_v, out_hbm.at[pl.ds(base + c * _CHUNK, _CHUNK)])


def _transpose_body(rows_ref, out_ref):
    for b in range(_TB):
        t = rows_ref[b].T
        out_ref[b, :, : _N] = t
        out_ref[b, :, _N:] = jnp.zeros((_D, _NP - _N), jnp.float32)


def _transpose(rows):
    return pl.pallas_call(
        _transpose_body,
        grid=(_B // _TB,),
        in_specs=[pl.BlockSpec((_TB, _N, _D), lambda b: (b, 0, 0))],
        out_specs=pl.BlockSpec((_TB, _D, _NP), lambda b: (b, 0, 0)),
        out_shape=jax.ShapeDtypeStruct((_B, _D, _NP), jnp.float32),
    )(rows)


def kernel(x, embed):
    x2 = jnp.sum(x * x, axis=1).reshape(_B, 1, _N)
    e2 = jnp.sum(embed * embed, axis=-1).reshape(_K, 1)

    embed_ind = _compute_codes(x, embed, x2, e2)
    rows = _sc_gather(embed, embed_ind.reshape(_B * _N))
    quantize = _transpose(rows.reshape(_B, _N, _D))[:, :, : _N]
    return (quantize, embed_ind)


# speedup vs baseline: 1.2187x; 1.0096x over previous
import functools

import jax
import jax.numpy as jnp
from jax import lax
from jax.experimental import pallas as pl
from jax.experimental.pallas import tpu as pltpu
from jax.experimental.pallas import tpu_sc as plsc

_B, _D, _N = 32, 256, 576
_K = 1024
_NP = 640
_H = _B // 2

_info = plsc.get_sparse_core_info()
_NC, _NS = _info.num_cores, _info.num_subcores
_NW = _NC * _NS
_RPW = (_H * _N) // _NW
_TB = 4


def _argmin_body(x_ref, embed_ref, x2_ref, e2_ref, idx_ref):
    xb = x_ref[0]
    scores = lax.dot_general(
        embed_ref[...], xb,
        dimension_numbers=(((1,), (0,)), ((), ())),
        precision=lax.Precision.DEFAULT,
        preferred_element_type=jnp.float32)
    q = (x2_ref[0] - 2.0 * scores) + e2_ref[...]
    minq = jnp.min(q, axis=0, keepdims=True)
    mc = jnp.maximum(minq, 0.0)
    sm = jnp.sqrt(mc)
    mi = lax.bitcast_convert_type(mc, jnp.int32)
    h = mc
    for delta in (1, 2, 3, 4):
        c = lax.bitcast_convert_type(mi + delta, jnp.float32)
        h = jnp.where(jnp.sqrt(c) == sm, c, h)
    iota = lax.broadcasted_iota(jnp.int32, (_K, _N), 0).astype(jnp.float32)
    cand = jnp.where(q <= h, iota, float(_K))
    idx_ref[0] = jnp.min(cand, axis=0, keepdims=True).astype(jnp.int32)


def _codes_half(x, embed, x2, e2, half):
    off = half * _H
    return pl.pallas_call(
        _argmin_body,
        grid=(_H,),
        in_specs=[
            pl.BlockSpec((1, _D, _N), lambda b: (b + off, 0, 0)),
            pl.BlockSpec((_K, _D), lambda b: (0, 0)),
            pl.BlockSpec((1, 1, _N), lambda b: (b + off, 0, 0)),
            pl.BlockSpec((_K, 1), lambda b: (0, 0)),
        ],
        out_specs=pl.BlockSpec((1, 1, _N), lambda b: (b, 0, 0)),
        out_shape=jax.ShapeDtypeStruct((_H, 1, _N), jnp.int32),
    )(x, embed, x2, e2)


@functools.partial(
    pl.kernel,
    mesh=plsc.VectorSubcoreMesh(core_axis_name="c", subcore_axis_name="s"),
    out_type=jax.ShapeDtypeStruct((_H * _N, _D), jnp.float32),
    scratch_types=[
        pltpu.VMEM((_RPW,), jnp.int32),
        pltpu.VMEM((_RPW, _D), jnp.float32),
        pltpu.SemaphoreType.DMA,
    ],
)
def _sc_gather(table_hbm, idx_hbm, out_hbm, idx_v, rows_v, sem):
    wid = lax.axis_index("s") * _NC + lax.axis_index("c")
    base = wid * _RPW
    pltpu.sync_copy(idx_hbm.at[pl.ds(base, _RPW)], idx_v)
    pltpu.async_copy(table_hbm.at[idx_v], rows_v, sem).wait()
    pltpu.sync_copy(rows_v, out_hbm.at[pl.ds(base, _RPW)])


def _transpose_body(r0_ref, r1_ref, i0_ref, i1_ref, out_ref, ind_ref):
    g = pl.program_id(0)
    zpad = jnp.zeros((_D, _NP - _N), jnp.float32)

    @pl.when(g < _H // _TB)
    def _lo():
        for b in range(_TB):
            out_ref[b, :, : _N] = r0_ref[b].T
            out_ref[b, :, _N:] = zpad
        ind_ref[...] = i0_ref[...]

    @pl.when(g >= _H // _TB)
    def _hi():
        for b in range(_TB):
            out_ref[b, :, : _N] = r1_ref[b].T
            out_ref[b, :, _N:] = zpad
        ind_ref[...] = i1_ref[...]


def _decode(rows0, rows1, idx0, idx1):
    hs = _H // _TB
    quant, ind = pl.pallas_call(
        _transpose_body,
        grid=(_B // _TB,),
        in_specs=[
            pl.BlockSpec((_TB, _N, _D), lambda g: (jnp.minimum(g, hs - 1), 0, 0)),
            pl.BlockSpec((_TB, _N, _D), lambda g: (jnp.maximum(g - hs, 0), 0, 0)),
            pl.BlockSpec((_TB, 1, _N), lambda g: (jnp.minimum(g, hs - 1), 0, 0)),
            pl.BlockSpec((_TB, 1, _N), lambda g: (jnp.maximum(g - hs, 0), 0, 0)),
        ],
        out_specs=[
            pl.BlockSpec((_TB, _D, _NP), lambda g: (g, 0, 0)),
            pl.BlockSpec((_TB, 1, _N), lambda g: (g, 0, 0)),
        ],
        out_shape=[
            jax.ShapeDtypeStruct((_B, _D, _NP), jnp.float32),
            jax.ShapeDtypeStruct((_B, 1, _N), jnp.int32),
        ],
    )(rows0, rows1, idx0, idx1)
    return quant, ind


def kernel(x, embed):
    x2 = jnp.sum(x * x, axis=1).reshape(_B, 1, _N)
    e2 = jnp.sum(embed * embed, axis=-1).reshape(_K, 1)

    idx0 = _codes_half(x, embed, x2, e2, 0)
    idx1 = _codes_half(x, embed, x2, e2, 1)
    rows0 = _sc_gather(embed, idx0.reshape(_H * _N))
    rows1 = _sc_gather(embed, idx1.reshape(_H * _N))
    quant, ind = _decode(rows0.reshape(_H, _N, _D), rows1.reshape(_H, _N, _D),
                         idx0, idx1)
    return (quant[:, :, : _N], ind.reshape(_B, _N))
